# baseline (speedup 1.0000x reference)
import functools

import jax
import jax.numpy as jnp
from jax import lax
from jax.experimental import pallas as pl
from jax.experimental.pallas import tpu as pltpu
from jax.experimental.pallas import tpu_sc as plsc

_N = 10000
_E = 320000
_H = 128
_DD = 64

_NC = 2
_NS = 16
_NW = _NC * _NS

_GCH = 128
_KS = 80
_E2 = _NW * _KS * _GCH

_NPAD = 10240

_EBLK = 2560
_NEB = _E2 // _EBLK
_NEB_REAL = _E // _EBLK


def _mesh():
    return plsc.VectorSubcoreMesh(
        core_axis_name="c", subcore_axis_name="s",
        num_cores=_NC, num_subcores=_NS,
    )


def _silu(x):
    return x * jax.nn.sigmoid(x)


def _node_mlp_body(h_ref, wl1_ref, bl1_ref, wl2_ref, bl2_ref,
                   wr1_ref, br1_ref, wr2_ref, br2_ref, hl_ref, hr_ref):
    hb = h_ref[...]
    x = jnp.dot(hb, wl1_ref[...], preferred_element_type=jnp.float32) + bl1_ref[...]
    hl_ref[...] = (
        jnp.dot(_silu(x), wl2_ref[...], preferred_element_type=jnp.float32)
        + bl2_ref[...]
    )
    y = jnp.dot(hb, wr1_ref[...], preferred_element_type=jnp.float32) + br1_ref[...]
    hr_ref[...] = (
        jnp.dot(_silu(y), wr2_ref[...], preferred_element_type=jnp.float32)
        + br2_ref[...]
    )


def _node_mlp(h, wl1, bl1, wl2, bl2, wr1, br1, wr2, br2):
    bn = 2000
    grid = (_N // bn,)
    row_spec = pl.BlockSpec((bn, _H), lambda i: (i, 0))
    full = lambda s: pl.BlockSpec(s, lambda i: (0, 0))
    return pl.pallas_call(
        _node_mlp_body,
        grid=grid,
        in_specs=[
            row_spec,
            full((_H, 2 * _H)), full((1, 2 * _H)), full((2 * _H, _H)), full((1, _H)),
            full((_H, 2 * _H)), full((1, 2 * _H)), full((2 * _H, _H)), full((1, _H)),
        ],
        out_specs=[row_spec, row_spec],
        out_shape=[
            jax.ShapeDtypeStruct((_N, _H), jnp.float32),
            jax.ShapeDtypeStruct((_N, _H), jnp.float32),
        ],
    )(h, wl1, bl1, wl2, bl2, wr1, br1, wr2, br2)


def _sc_gather_body(hl_hbm, hr_hbm, px_hbm, py_hbm, pz_hbm, row_hbm, col_hbm,
                    hlg_hbm, hrg_hbm, rel_hbm,
                    idxr_v, idxc_v, bufl_v, bufr_v, relbuf_v,
                    px_v, py_v, pz_v,
                    semi0, semi1, semg0, semg1, semw0, semw1, semr0, semr1):
    wid = lax.axis_index("s") * _NC + lax.axis_index("c")

    pltpu.sync_copy(px_hbm, px_v)
    pltpu.sync_copy(py_hbm, py_v)
    pltpu.sync_copy(pz_hbm, pz_v)

    zero = jnp.zeros((16,), jnp.float32)
    for j in range(2):
        for i in range(_GCH):
            relbuf_v[j, i] = zero

    lanes = lax.iota(jnp.int32, 16)
    semi = (semi0, semi1)
    semg = (semg0, semg1)
    semw = (semw0, semw1)
    semr = (semr0, semr1)
    cols = (jnp.zeros((16,), jnp.int32), jnp.ones((16,), jnp.int32),
            jnp.full((16,), 2, jnp.int32))
    slv = (jnp.zeros((16,), jnp.int32), jnp.ones((16,), jnp.int32))

    def cbase(k):
        return (k * _NW + wid) * _GCH

    pltpu.async_copy(row_hbm.at[pl.ds(cbase(0), _GCH)], idxr_v.at[0], semi0)
    pltpu.async_copy(col_hbm.at[pl.ds(cbase(0), _GCH)], idxc_v.at[0], semi0)

    def body2(k2, carry):
        for sl in range(2):
            k = k2 * 2 + sl
            base = cbase(k)
            pltpu.make_async_copy(
                row_hbm.at[pl.ds(base, _GCH)], idxr_v.at[sl], semi[sl]).wait()
            pltpu.make_async_copy(
                col_hbm.at[pl.ds(base, _GCH)], idxc_v.at[sl], semi[sl]).wait()

            @pl.when(k < _KS - 1)
            def _():
                nb = cbase(k + 1)
                pltpu.async_copy(
                    row_hbm.at[pl.ds(nb, _GCH)], idxr_v.at[1 - sl], semi[1 - sl])
                pltpu.async_copy(
                    col_hbm.at[pl.ds(nb, _GCH)], idxc_v.at[1 - sl], semi[1 - sl])

            @pl.when(k2 > 0)
            def _():
                pltpu.make_async_copy(
                    bufl_v.at[sl], hlg_hbm.at[pl.ds(base, _GCH)], semw[sl]).wait()
                pltpu.make_async_copy(
                    bufr_v.at[sl], hrg_hbm.at[pl.ds(base, _GCH)], semw[sl]).wait()
                pltpu.make_async_copy(
                    relbuf_v.at[sl], rel_hbm.at[pl.ds(base, _GCH)], semr[sl]).wait()

            pltpu.async_copy(hl_hbm.at[idxr_v.at[sl]], bufl_v.at[sl], semg[sl])
            pltpu.async_copy(hr_hbm.at[idxc_v.at[sl]], bufr_v.at[sl], semg[sl])

            for g in range(_GCH // 16):
                ir = idxr_v[sl, pl.ds(g * 16, 16)]
                ic = idxc_v[sl, pl.ds(g * 16, 16)]
                rows = g * 16 + lanes
                rx = plsc.load_gather(px_v, [ir]) - plsc.load_gather(px_v, [ic])
                plsc.store_scatter(relbuf_v, [slv[sl], rows, cols[0]], rx)
                ry = plsc.load_gather(py_v, [ir]) - plsc.load_gather(py_v, [ic])
                plsc.store_scatter(relbuf_v, [slv[sl], rows, cols[1]], ry)
                rz = plsc.load_gather(pz_v, [ir]) - plsc.load_gather(pz_v, [ic])
                plsc.store_scatter(relbuf_v, [slv[sl], rows, cols[2]], rz)
            pltpu.async_copy(relbuf_v.at[sl], rel_hbm.at[pl.ds(base, _GCH)], semr[sl])

            pltpu.make_async_copy(
                hl_hbm.at[idxr_v.at[sl]], bufl_v.at[sl], semg[sl]).wait()
            pltpu.make_async_copy(
                hr_hbm.at[idxc_v.at[sl]], bufr_v.at[sl], semg[sl]).wait()
            pltpu.async_copy(bufl_v.at[sl], hlg_hbm.at[pl.ds(base, _GCH)], semw[sl])
            pltpu.async_copy(bufr_v.at[sl], hrg_hbm.at[pl.ds(base, _GCH)], semw[sl])
        return carry

    lax.fori_loop(0, _KS // 2, body2, 0)

    for sl in range(2):
        k = _KS - 2 + sl
        base = cbase(k)
        pltpu.make_async_copy(
            bufl_v.at[sl], hlg_hbm.at[pl.ds(base, _GCH)], semw[sl]).wait()
        pltpu.make_async_copy(
            bufr_v.at[sl], hrg_hbm.at[pl.ds(base, _GCH)], semw[sl]).wait()
        pltpu.make_async_copy(
            relbuf_v.at[sl], rel_hbm.at[pl.ds(base, _GCH)], semr[sl]).wait()


@functools.lru_cache(maxsize=None)
def _make_sc_gather():
    return pl.kernel(
        _sc_gather_body,
        out_type=(
            jax.ShapeDtypeStruct((_E2, _H), jnp.float32),
            jax.ShapeDtypeStruct((_E2, _H), jnp.float32),
            jax.ShapeDtypeStruct((_E2, 16), jnp.float32),
        ),
        mesh=_mesh(),
        compiler_params=pltpu.CompilerParams(needs_layout_passes=False),
        scratch_types=[
            pltpu.VMEM((2, _GCH), jnp.int32),
            pltpu.VMEM((2, _GCH), jnp.int32),
            pltpu.VMEM((2, _GCH, _H), jnp.float32),
            pltpu.VMEM((2, _GCH, _H), jnp.float32),
            pltpu.VMEM((2, _GCH, 16), jnp.float32),
            pltpu.VMEM((_N,), jnp.float32),
            pltpu.VMEM((_N,), jnp.float32),
            pltpu.VMEM((_N,), jnp.float32),
            pltpu.SemaphoreType.DMA,
            pltpu.SemaphoreType.DMA,
            pltpu.SemaphoreType.DMA,
            pltpu.SemaphoreType.DMA,
            pltpu.SemaphoreType.DMA,
            pltpu.SemaphoreType.DMA,
            pltpu.SemaphoreType.DMA,
            pltpu.SemaphoreType.DMA,
        ],
    )


def _sc_gather(hl, hr, px, py, pz, row, col):
    return _make_sc_gather()(hl, hr, px, py, pz, row, col)


def _edge_body(hlg_ref, hrg_ref, rel_ref, ea_ref, tm_ref,
               wea_ref, wed_ref, be_ref, wn_ref, bn_ref,
               wi1a_ref, wi1t_ref, bi1_ref, wi2_ref, bi2_ref, out_ref):
    rel = rel_ref[...]
    d2 = jnp.sum(rel * rel, axis=1, keepdims=True)
    dist = jnp.sqrt(d2)
    b = rel.shape[0]
    step = 15.0 / (_DD - 1)
    coeff = -0.5 / (step * step)
    offs = lax.broadcasted_iota(jnp.int32, (b, _DD), 1).astype(jnp.float32) * step
    demb = jnp.exp(coeff * (dist - offs) ** 2)
    ea = (
        jnp.dot(ea_ref[...], wea_ref[...], preferred_element_type=jnp.float32)
        + jnp.dot(demb, wed_ref[...], preferred_element_type=jnp.float32)
        + be_ref[...]
    )
    nf = (
        jnp.dot(hlg_ref[...] * hrg_ref[...], wn_ref[...],
                preferred_element_type=jnp.float32)
        + bn_ref[...]
    )
    x = (
        jnp.dot(ea * nf, wi1a_ref[...], preferred_element_type=jnp.float32)
        + jnp.dot(tm_ref[...], wi1t_ref[...], preferred_element_type=jnp.float32)
        + bi1_ref[...]
    )
    t1 = _silu(x)
    inter = jnp.sum(t1 * wi2_ref[...], axis=1, keepdims=True) + bi2_ref[...]
    w = inter / ((dist + 1.0) * jnp.maximum(dist, 1e-12))
    real = pl.program_id(0) < _NEB_REAL
    out_ref[...] = jnp.where(real, rel * w, 0.0)


def _edge_compute(hlg, hrg, rel, edge_attr, tm,
                  wea, wed, be, wn, bn, wi1a, wi1t, bi1, wi2, bi2):
    grid = (_NEB,)
    rs2 = lambda w: pl.BlockSpec((_EBLK, w), lambda i: (i, 0))
    rsc = lambda w: pl.BlockSpec(
        (_EBLK, w), lambda i: (jnp.minimum(i, _NEB_REAL - 1), 0))
    full = lambda s: pl.BlockSpec(s, lambda i: (0,) * len(s))
    return pl.pallas_call(
        _edge_body,
        grid=grid,
        in_specs=[
            rs2(_H), rs2(_H), rs2(16), rsc(16), rsc(16),
            full((16, _H)), full((_DD, _H)), full((1, _H)),
            full((_H, _H)), full((1, _H)),
            full((_H, 2 * _H)), full((16, 2 * _H)), full((1, 2 * _H)),
            full((1, 2 * _H)), full((1, 1)),
        ],
        out_specs=rs2(16),
        out_shape=jax.ShapeDtypeStruct((_E2, 16), jnp.float32),
    )(hlg, hrg, rel, edge_attr, tm,
      wea, wed, be, wn, bn, wi1a, wi1t, bi1, wi2, bi2)


def _sc_scatter_body(force_hbm, row_hbm, zeros_hbm, out_hbm,
                     val_v, idx_v, accx_v, accy_v, accz_v, semv0, semv1):
    c = lax.axis_index("c")
    s = lax.axis_index("s")
    wid = s * _NC + c
    pltpu.sync_copy(zeros_hbm, accx_v)
    pltpu.sync_copy(zeros_hbm, accy_v)
    pltpu.sync_copy(zeros_hbm, accz_v)

    lanes = lax.iota(jnp.int32, 16)
    col0 = jnp.zeros((16,), jnp.int32)
    col1 = jnp.ones((16,), jnp.int32)
    col2 = jnp.full((16,), 2, jnp.int32)
    slv = (jnp.zeros((16,), jnp.int32), jnp.ones((16,), jnp.int32))
    semv = (semv0, semv1)

    def cbase(k):
        return (k * _NW + wid) * _GCH

    pltpu.async_copy(row_hbm.at[pl.ds(cbase(0), _GCH)], idx_v.at[0], semv0)
    pltpu.async_copy(force_hbm.at[pl.ds(cbase(0), _GCH)], val_v.at[0], semv0)

    def body2(k2, carry):
        for sl in range(2):
            k = k2 * 2 + sl
            base = cbase(k)
            pltpu.make_async_copy(
                row_hbm.at[pl.ds(base, _GCH)], idx_v.at[sl], semv[sl]).wait()
            pltpu.make_async_copy(
                force_hbm.at[pl.ds(base, _GCH)], val_v.at[sl], semv[sl]).wait()

            @pl.when(k < _KS - 1)
            def _():
                nb = cbase(k + 1)
                pltpu.async_copy(
                    row_hbm.at[pl.ds(nb, _GCH)], idx_v.at[1 - sl], semv[1 - sl])
                pltpu.async_copy(
                    force_hbm.at[pl.ds(nb, _GCH)], val_v.at[1 - sl], semv[1 - sl])

            for g in range(_GCH // 16):
                rows = g * 16 + lanes
                ir = idx_v[sl, pl.ds(g * 16, 16)]
                fx = plsc.load_gather(val_v, [slv[sl], rows, col0])
                plsc.addupdate_scatter(accx_v, [ir], fx)
                fy = plsc.load_gather(val_v, [slv[sl], rows, col1])
                plsc.addupdate_scatter(accy_v, [ir], fy)
                fz = plsc.load_gather(val_v, [slv[sl], rows, col2])
                plsc.addupdate_scatter(accz_v, [ir], fz)
        return carry

    lax.fori_loop(0, _KS // 2, body2, 0)
    obase = wid * 3 * _NPAD
    pltpu.sync_copy(accx_v, out_hbm.at[pl.ds(obase, _NPAD)])
    pltpu.sync_copy(accy_v, out_hbm.at[pl.ds(obase + _NPAD, _NPAD)])
    pltpu.sync_copy(accz_v, out_hbm.at[pl.ds(obase + 2 * _NPAD, _NPAD)])


@functools.lru_cache(maxsize=None)
def _make_sc_scatter():
    return pl.kernel(
        _sc_scatter_body,
        out_type=jax.ShapeDtypeStruct((_NW * 3 * _NPAD,), jnp.float32),
        mesh=_mesh(),
        compiler_params=pltpu.CompilerParams(needs_layout_passes=False),
        scratch_types=[
            pltpu.VMEM((2, _GCH, 16), jnp.float32),
            pltpu.VMEM((2, _GCH), jnp.int32),
            pltpu.VMEM((_NPAD,), jnp.float32),
            pltpu.VMEM((_NPAD,), jnp.float32),
            pltpu.VMEM((_NPAD,), jnp.float32),
            pltpu.SemaphoreType.DMA,
            pltpu.SemaphoreType.DMA,
        ],
    )


def _sc_scatter(force, row, zeros1):
    return _make_sc_scatter()(force, row, zeros1)


def _psum_body(p_ref, out_ref):
    acc = p_ref[0]
    for i in range(1, _NW):
        acc = acc + p_ref[i]
    out_ref[...] = acc


def _psum(p):
    npk = _NPAD // 16
    return pl.pallas_call(
        _psum_body,
        grid=(1,),
        in_specs=[pl.BlockSpec((_NW, npk, 128), lambda i: (0, 0, 0))],
        out_specs=pl.BlockSpec((npk, 128), lambda i: (0, 0)),
        out_shape=jax.ShapeDtypeStruct((npk, 128), jnp.float32),
    )(p)


def _final_body(d_ref, h_ref, ws1h_ref, ws1n_ref, bs1_ref,
                ws2_ref, bs2_ref, out_ref):
    delta = d_ref[...]
    nrm = jnp.sqrt(jnp.sum(delta * delta, axis=1, keepdims=True))
    x = (
        jnp.dot(h_ref[...], ws1h_ref[...], preferred_element_type=jnp.float32)
        + nrm * ws1n_ref[...]
        + bs1_ref[...]
    )
    t = _silu(x)
    g = jnp.sum(t * ws2_ref[...], axis=1, keepdims=True) + bs2_ref[...]
    scale = jax.nn.softplus(g)
    out_ref[...] = delta * scale


def _final(d8, h, ws1h, ws1n, bs1, ws2, bs2):
    bn = 2000
    grid = (_N // bn,)
    rs = lambda w: pl.BlockSpec((bn, w), lambda i: (i, 0))
    full = lambda s: pl.BlockSpec(s, lambda i: (0, 0))
    return pl.pallas_call(
        _final_body,
        grid=grid,
        in_specs=[
            rs(8), rs(_H),
            full((_H, _H // 2)), full((1, _H // 2)), full((1, _H // 2)),
            full((1, _H // 2)), full((1, 1)),
        ],
        out_specs=rs(8),
        out_shape=jax.ShapeDtypeStruct((_N, 8), jnp.float32),
    )(d8, h, ws1h, ws1n, bs1, ws2, bs2)


def kernel(pos, h, edge_index, edge_attr, time,
           W_nl1, b_nl1, W_nl2, b_nl2,
           W_nr1, b_nr1, W_nr2, b_nr2,
           W_e, b_e, W_n, b_n,
           W_i1, b_i1, W_i2, b_i2,
           W_s1, b_s1, W_s2, b_s2):
    row = edge_index[0].astype(jnp.int32)
    col = edge_index[1].astype(jnp.int32)

    row2 = jnp.pad(row, (0, _E2 - _E))
    col2 = jnp.pad(col, (0, _E2 - _E))
    px = pos[:, 0]
    py = pos[:, 1]
    pz = pos[:, 2]
    r2 = lambda v: v.reshape(1, -1)

    hl, hr = _node_mlp(h, W_nl1, r2(b_nl1), W_nl2, r2(b_nl2),
                       W_nr1, r2(b_nr1), W_nr2, r2(b_nr2))

    hlg, hrg, rel = _sc_gather(hl, hr, px, py, pz, row2, col2)

    force = _edge_compute(
        hlg, hrg, rel, edge_attr, time,
        W_e[:16], W_e[16:], r2(b_e), W_n, r2(b_n),
        W_i1[:_H], W_i1[_H:], r2(b_i1), W_i2.reshape(1, -1), b_i2.reshape(1, 1),
    )

    zeros1 = jnp.zeros((_NPAD,), jnp.float32)
    praw = _sc_scatter(force, row2, zeros1).reshape(_NW, 3, _NPAD)
    ppk = jnp.pad(praw.transpose(0, 2, 1), ((0, 0), (0, 0), (0, 5)))
    ppk = ppk.reshape(_NW, _NPAD // 16, 128)

    delta8 = _psum(ppk).reshape(_NPAD, 8)[:_N]

    out8 = _final(delta8, h,
                  W_s1[:_H], W_s1[_H:], r2(b_s1),
                  W_s2.reshape(1, -1), b_s2.reshape(1, 1))
    return out8[:, :3]


# speedup vs baseline: 1.2340x; 1.2340x over previous
import functools

import jax
import jax.numpy as jnp
from jax import lax
from jax.experimental import pallas as pl
from jax.experimental.pallas import tpu as pltpu
from jax.experimental.pallas import tpu_sc as plsc

_N = 10000
_E = 320000
_H = 128
_DD = 64

_NC = 2
_NS = 16
_NW = _NC * _NS

_GCH = 128
_NCHUNKS = _E // _GCH
_KFULL = _NCHUNKS // _NW
_KREM = _NCHUNKS % _NW

_NPAD = 10240
_NPT = _NPAD // _NS


def _mesh():
    return plsc.VectorSubcoreMesh(
        core_axis_name="c", subcore_axis_name="s",
        num_cores=_NC, num_subcores=_NS,
    )


def _silu(x):
    return x * jax.nn.sigmoid(x)


def _node_mlp_body(h_ref, wl1_ref, bl1_ref, wl2_ref, bl2_ref,
                   wr1_ref, br1_ref, wr2_ref, br2_ref, hl_ref, hr_ref):
    hb = h_ref[...]
    x = jnp.dot(hb, wl1_ref[...], preferred_element_type=jnp.float32) + bl1_ref[...]
    hl_ref[...] = (
        jnp.dot(_silu(x), wl2_ref[...], preferred_element_type=jnp.float32)
        + bl2_ref[...]
    )
    y = jnp.dot(hb, wr1_ref[...], preferred_element_type=jnp.float32) + br1_ref[...]
    hr_ref[...] = (
        jnp.dot(_silu(y), wr2_ref[...], preferred_element_type=jnp.float32)
        + br2_ref[...]
    )


def _node_mlp(h, wl1, bl1, wl2, bl2, wr1, br1, wr2, br2):
    bn = 2000
    grid = (_N // bn,)
    row_spec = pl.BlockSpec((bn, _H), lambda i: (i, 0))
    full = lambda s: pl.BlockSpec(s, lambda i: (0, 0))
    return pl.pallas_call(
        _node_mlp_body,
        grid=grid,
        in_specs=[
            row_spec,
            full((_H, 2 * _H)), full((1, 2 * _H)), full((2 * _H, _H)), full((1, _H)),
            full((_H, 2 * _H)), full((1, 2 * _H)), full((2 * _H, _H)), full((1, _H)),
        ],
        out_specs=[row_spec, row_spec],
        out_shape=[
            jax.ShapeDtypeStruct((_N, _H), jnp.float32),
            jax.ShapeDtypeStruct((_N, _H), jnp.float32),
        ],
    )(h, wl1, bl1, wl2, bl2, wr1, br1, wr2, br2)


def _sc_gather_body(hl_hbm, hr_hbm, px_hbm, py_hbm, pz_hbm, row_hbm, col_hbm,
                    hlg_hbm, hrg_hbm, rel_hbm,
                    idxr_v, idxc_v, bufl_v, bufr_v, relbuf_v,
                    px_v, py_v, pz_v,
                    sem1, sem2, sem3):
    wid = lax.axis_index("s") * _NC + lax.axis_index("c")
    n_k = _KFULL + jnp.where(wid < _KREM, 1, 0)

    pltpu.sync_copy(px_hbm, px_v)
    pltpu.sync_copy(py_hbm, py_v)
    pltpu.sync_copy(pz_hbm, pz_v)

    zero = jnp.zeros((16,), jnp.float32)
    for i in range(_GCH):
        relbuf_v[i] = zero

    lanes = lax.iota(jnp.int32, 16)

    def body(k, carry):
        base = (k * _NW + wid) * _GCH
        i1 = pltpu.async_copy(row_hbm.at[pl.ds(base, _GCH)], idxr_v, sem1)
        i2 = pltpu.async_copy(col_hbm.at[pl.ds(base, _GCH)], idxc_v, sem2)
        i1.wait()
        i2.wait()
        c1 = pltpu.async_copy(hl_hbm.at[idxr_v], bufl_v, sem1)
        c2 = pltpu.async_copy(hr_hbm.at[idxc_v], bufr_v, sem2)
        for g in range(_GCH // 16):
            ir = idxr_v[pl.ds(g * 16, 16)]
            ic = idxc_v[pl.ds(g * 16, 16)]
            rows = g * 16 + lanes
            rx = plsc.load_gather(px_v, [ir]) - plsc.load_gather(px_v, [ic])
            plsc.store_scatter(relbuf_v, [rows, jnp.zeros((16,), jnp.int32)], rx)
            ry = plsc.load_gather(py_v, [ir]) - plsc.load_gather(py_v, [ic])
            plsc.store_scatter(relbuf_v, [rows, jnp.ones((16,), jnp.int32)], ry)
            rz = plsc.load_gather(pz_v, [ir]) - plsc.load_gather(pz_v, [ic])
            plsc.store_scatter(relbuf_v, [rows, jnp.full((16,), 2, jnp.int32)], rz)
        c3 = pltpu.async_copy(relbuf_v, rel_hbm.at[pl.ds(base, _GCH)], sem3)
        c1.wait()
        c2.wait()
        pltpu.sync_copy(bufl_v, hlg_hbm.at[pl.ds(base, _GCH)])
        pltpu.sync_copy(bufr_v, hrg_hbm.at[pl.ds(base, _GCH)])
        c3.wait()
        return carry

    lax.fori_loop(0, n_k, body, 0)


@functools.lru_cache(maxsize=None)
def _make_sc_gather():
    return pl.kernel(
        _sc_gather_body,
        out_type=(
            jax.ShapeDtypeStruct((_E, _H), jnp.float32),
            jax.ShapeDtypeStruct((_E, _H), jnp.float32),
            jax.ShapeDtypeStruct((_E, 16), jnp.float32),
        ),
        mesh=_mesh(),
        compiler_params=pltpu.CompilerParams(needs_layout_passes=False),
        scratch_types=[
            pltpu.VMEM((_GCH,), jnp.int32),
            pltpu.VMEM((_GCH,), jnp.int32),
            pltpu.VMEM((_GCH, _H), jnp.float32),
            pltpu.VMEM((_GCH, _H), jnp.float32),
            pltpu.VMEM((_GCH, 16), jnp.float32),
            pltpu.VMEM((_N,), jnp.float32),
            pltpu.VMEM((_N,), jnp.float32),
            pltpu.VMEM((_N,), jnp.float32),
            pltpu.SemaphoreType.DMA,
            pltpu.SemaphoreType.DMA,
            pltpu.SemaphoreType.DMA,
        ],
    )


def _sc_gather(hl, hr, px, py, pz, row, col):
    return _make_sc_gather()(hl, hr, px, py, pz, row, col)


def _edge_body(hlg_ref, hrg_ref, rel_ref, ea_ref, tm_ref,
               wea_ref, wed_ref, be_ref, wn_ref, bn_ref,
               wi1a_ref, wi1t_ref, bi1_ref, wi2_ref, bi2_ref, out_ref):
    rel = rel_ref[...]
    d2 = jnp.sum(rel * rel, axis=1, keepdims=True)
    dist = jnp.sqrt(d2)
    b = rel.shape[0]
    step = 15.0 / (_DD - 1)
    coeff = -0.5 / (step * step)
    offs = lax.broadcasted_iota(jnp.int32, (b, _DD), 1).astype(jnp.float32) * step
    demb = jnp.exp(coeff * (dist - offs) ** 2)
    ea = (
        jnp.dot(ea_ref[...], wea_ref[...], preferred_element_type=jnp.float32)
        + jnp.dot(demb, wed_ref[...], preferred_element_type=jnp.float32)
        + be_ref[...]
    )
    nf = (
        jnp.dot(hlg_ref[...] * hrg_ref[...], wn_ref[...],
                preferred_element_type=jnp.float32)
        + bn_ref[...]
    )
    x = (
        jnp.dot(ea * nf, wi1a_ref[...], preferred_element_type=jnp.float32)
        + jnp.dot(tm_ref[...], wi1t_ref[...], preferred_element_type=jnp.float32)
        + bi1_ref[...]
    )
    t1 = _silu(x)
    inter = jnp.sum(t1 * wi2_ref[...], axis=1, keepdims=True) + bi2_ref[...]
    w = inter / ((dist + 1.0) * jnp.maximum(dist, 1e-12))
    out_ref[...] = rel * w


def _edge_compute(hlg, hrg, rel, edge_attr, tm,
                  wea, wed, be, wn, bn, wi1a, wi1t, bi1, wi2, bi2):
    be_blk = 2560
    grid = (_E // be_blk,)
    rs = lambda w: pl.BlockSpec((be_blk, w), lambda i: (i, 0))
    full = lambda s: pl.BlockSpec(s, lambda i: (0,) * len(s))
    return pl.pallas_call(
        _edge_body,
        grid=grid,
        in_specs=[
            rs(_H), rs(_H), rs(16), rs(16), rs(16),
            full((16, _H)), full((_DD, _H)), full((1, _H)),
            full((_H, _H)), full((1, _H)),
            full((_H, 2 * _H)), full((16, 2 * _H)), full((1, 2 * _H)),
            full((1, 2 * _H)), full((1, 1)),
        ],
        out_specs=rs(16),
        out_shape=jax.ShapeDtypeStruct((_E, 16), jnp.float32),
    )(hlg, hrg, rel, edge_attr, tm,
      wea, wed, be, wn, bn, wi1a, wi1t, bi1, wi2, bi2)


def _sc_scatter_body(force_hbm, row_hbm, zeros_hbm, out_hbm,
                     val_v, idx_v, accx_v, accy_v, accz_v, sem, sem2):
    c = lax.axis_index("c")
    s = lax.axis_index("s")
    wid = s * _NC + c
    pltpu.sync_copy(zeros_hbm, accx_v)
    pltpu.sync_copy(zeros_hbm, accy_v)
    pltpu.sync_copy(zeros_hbm, accz_v)

    lanes = lax.iota(jnp.int32, 16)
    col0 = jnp.zeros((16,), jnp.int32)
    col1 = jnp.ones((16,), jnp.int32)
    col2 = jnp.full((16,), 2, jnp.int32)

    n_k = _KFULL + jnp.where(wid < _KREM, 1, 0)

    def body(k, carry):
        base = (k * _NW + wid) * _GCH
        i1 = pltpu.async_copy(row_hbm.at[pl.ds(base, _GCH)], idx_v, sem)
        i2 = pltpu.async_copy(force_hbm.at[pl.ds(base, _GCH)], val_v, sem2)
        i1.wait()
        i2.wait()
        for g in range(_GCH // 16):
            rows = g * 16 + lanes
            ir = idx_v[pl.ds(g * 16, 16)]
            fx = plsc.load_gather(val_v, [rows, col0])
            plsc.addupdate_scatter(accx_v, [ir], fx)
            fy = plsc.load_gather(val_v, [rows, col1])
            plsc.addupdate_scatter(accy_v, [ir], fy)
            fz = plsc.load_gather(val_v, [rows, col2])
            plsc.addupdate_scatter(accz_v, [ir], fz)
        return carry

    lax.fori_loop(0, n_k, body, 0)
    obase = wid * 3 * _NPAD
    pltpu.sync_copy(accx_v, out_hbm.at[pl.ds(obase, _NPAD)])
    pltpu.sync_copy(accy_v, out_hbm.at[pl.ds(obase + _NPAD, _NPAD)])
    pltpu.sync_copy(accz_v, out_hbm.at[pl.ds(obase + 2 * _NPAD, _NPAD)])


@functools.lru_cache(maxsize=None)
def _make_sc_scatter():
    return pl.kernel(
        _sc_scatter_body,
        out_type=jax.ShapeDtypeStruct((_NW * 3 * _NPAD,), jnp.float32),
        mesh=_mesh(),
        compiler_params=pltpu.CompilerParams(needs_layout_passes=False),
        scratch_types=[
            pltpu.VMEM((_GCH, 16), jnp.float32),
            pltpu.VMEM((_GCH,), jnp.int32),
            pltpu.VMEM((_NPAD,), jnp.float32),
            pltpu.VMEM((_NPAD,), jnp.float32),
            pltpu.VMEM((_NPAD,), jnp.float32),
            pltpu.SemaphoreType.DMA,
            pltpu.SemaphoreType.DMA,
        ],
    )


def _sc_scatter(force, row, zeros1):
    return _make_sc_scatter()(force, row, zeros1)


def _psum_body(p_ref, out_ref):
    acc = p_ref[0]
    for i in range(1, _NW):
        acc = acc + p_ref[i]
    out_ref[...] = acc


def _psum(p):
    npk = _NPAD // 16
    return pl.pallas_call(
        _psum_body,
        grid=(1,),
        in_specs=[pl.BlockSpec((_NW, npk, 128), lambda i: (0, 0, 0))],
        out_specs=pl.BlockSpec((npk, 128), lambda i: (0, 0)),
        out_shape=jax.ShapeDtypeStruct((npk, 128), jnp.float32),
    )(p)


def _final_body(d_ref, h_ref, ws1h_ref, ws1n_ref, bs1_ref,
                ws2_ref, bs2_ref, out_ref):
    delta = d_ref[...]
    nrm = jnp.sqrt(jnp.sum(delta * delta, axis=1, keepdims=True))
    x = (
        jnp.dot(h_ref[...], ws1h_ref[...], preferred_element_type=jnp.float32)
        + nrm * ws1n_ref[...]
        + bs1_ref[...]
    )
    t = _silu(x)
    g = jnp.sum(t * ws2_ref[...], axis=1, keepdims=True) + bs2_ref[...]
    scale = jax.nn.softplus(g)
    out_ref[...] = delta * scale


def _final(d8, h, ws1h, ws1n, bs1, ws2, bs2):
    bn = 2000
    grid = (_N // bn,)
    rs = lambda w: pl.BlockSpec((bn, w), lambda i: (i, 0))
    full = lambda s: pl.BlockSpec(s, lambda i: (0, 0))
    return pl.pallas_call(
        _final_body,
        grid=grid,
        in_specs=[
            rs(8), rs(_H),
            full((_H, _H // 2)), full((1, _H // 2)), full((1, _H // 2)),
            full((1, _H // 2)), full((1, 1)),
        ],
        out_specs=rs(8),
        out_shape=jax.ShapeDtypeStruct((_N, 8), jnp.float32),
    )(d8, h, ws1h, ws1n, bs1, ws2, bs2)


def kernel(pos, h, edge_index, edge_attr, time,
           W_nl1, b_nl1, W_nl2, b_nl2,
           W_nr1, b_nr1, W_nr2, b_nr2,
           W_e, b_e, W_n, b_n,
           W_i1, b_i1, W_i2, b_i2,
           W_s1, b_s1, W_s2, b_s2):
    row = edge_index[0].astype(jnp.int32)
    col = edge_index[1].astype(jnp.int32)

    px = pos[:, 0]
    py = pos[:, 1]
    pz = pos[:, 2]
    r2 = lambda v: v.reshape(1, -1)

    hl, hr = _node_mlp(h, W_nl1, r2(b_nl1), W_nl2, r2(b_nl2),
                       W_nr1, r2(b_nr1), W_nr2, r2(b_nr2))

    hlg, hrg, rel = _sc_gather(hl, hr, px, py, pz, row, col)

    force = _edge_compute(
        hlg, hrg, rel, edge_attr, time,
        W_e[:16], W_e[16:], r2(b_e), W_n, r2(b_n),
        W_i1[:_H], W_i1[_H:], r2(b_i1), W_i2.reshape(1, -1), b_i2.reshape(1, 1),
    )

    zeros1 = jnp.zeros((_NPAD,), jnp.float32)
    praw = _sc_scatter(force, row, zeros1).reshape(_NW, 3, _NPAD)
    ppk = jnp.pad(praw.transpose(0, 2, 1), ((0, 0), (0, 0), (0, 5)))
    ppk = ppk.reshape(_NW, _NPAD // 16, 128)

    delta8 = _psum(ppk).reshape(_NPAD, 8)[:_N]

    out8 = _final(delta8, h,
                  W_s1[:_H], W_s1[_H:], r2(b_s1),
                  W_s2.reshape(1, -1), b_s2.reshape(1, 1))
    return out8[:, :3]


# speedup vs baseline: 1.2365x; 1.0020x over previous
import functools

import jax
import jax.numpy as jnp
from jax import lax
from jax.experimental import pallas as pl
from jax.experimental.pallas import tpu as pltpu
from jax.experimental.pallas import tpu_sc as plsc

_N = 10000
_E = 320000
_H = 128
_DD = 64

_NC = 2
_NS = 16
_NW = _NC * _NS

_GCH = 128
_NCHUNKS = _E // _GCH
_KFULL = _NCHUNKS // _NW
_KREM = _NCHUNKS % _NW

_NPAD = 10240
_NPT = _NPAD // _NS


def _mesh():
    return plsc.VectorSubcoreMesh(
        core_axis_name="c", subcore_axis_name="s",
        num_cores=_NC, num_subcores=_NS,
    )


def _silu(x):
    return x * jax.nn.sigmoid(x)


def _node_mlp_body(h_ref, wl1_ref, bl1_ref, wl2_ref, bl2_ref,
                   wr1_ref, br1_ref, wr2_ref, br2_ref, hl_ref, hr_ref):
    hb = h_ref[...]
    x = jnp.dot(hb, wl1_ref[...], preferred_element_type=jnp.float32) + bl1_ref[...]
    hl_ref[...] = (
        jnp.dot(_silu(x), wl2_ref[...], preferred_element_type=jnp.float32)
        + bl2_ref[...]
    )
    y = jnp.dot(hb, wr1_ref[...], preferred_element_type=jnp.float32) + br1_ref[...]
    hr_ref[...] = (
        jnp.dot(_silu(y), wr2_ref[...], preferred_element_type=jnp.float32)
        + br2_ref[...]
    )


def _node_mlp(h, wl1, bl1, wl2, bl2, wr1, br1, wr2, br2):
    bn = 2000
    grid = (_N // bn,)
    row_spec = pl.BlockSpec((bn, _H), lambda i: (i, 0))
    full = lambda s: pl.BlockSpec(s, lambda i: (0, 0))
    return pl.pallas_call(
        _node_mlp_body,
        grid=grid,
        in_specs=[
            row_spec,
            full((_H, 2 * _H)), full((1, 2 * _H)), full((2 * _H, _H)), full((1, _H)),
            full((_H, 2 * _H)), full((1, 2 * _H)), full((2 * _H, _H)), full((1, _H)),
        ],
        out_specs=[row_spec, row_spec],
        out_shape=[
            jax.ShapeDtypeStruct((_N, _H), jnp.float32),
            jax.ShapeDtypeStruct((_N, _H), jnp.float32),
        ],
    )(h, wl1, bl1, wl2, bl2, wr1, br1, wr2, br2)


def _sc_gather_body(hl_hbm, hr_hbm, px_hbm, py_hbm, pz_hbm, row_hbm, col_hbm,
                    hlg_hbm, hrg_hbm, rel_hbm,
                    idxr_v, idxc_v, bufl_v, bufr_v, relbuf_v,
                    px_v, py_v, pz_v,
                    sem1, sem2, sem3):
    wid = lax.axis_index("s") * _NC + lax.axis_index("c")
    n_k = _KFULL + jnp.where(wid < _KREM, 1, 0)

    pltpu.sync_copy(px_hbm, px_v)
    pltpu.sync_copy(py_hbm, py_v)
    pltpu.sync_copy(pz_hbm, pz_v)

    zero = jnp.zeros((16,), jnp.float32)
    for i in range(_GCH):
        relbuf_v[i] = zero

    lanes = lax.iota(jnp.int32, 16)

    def body(k, carry):
        base = (k * _NW + wid) * _GCH
        i1 = pltpu.async_copy(row_hbm.at[pl.ds(base, _GCH)], idxr_v, sem1)
        i2 = pltpu.async_copy(col_hbm.at[pl.ds(base, _GCH)], idxc_v, sem2)
        i1.wait()
        i2.wait()
        c1 = pltpu.async_copy(hl_hbm.at[idxr_v], bufl_v, sem1)
        c2 = pltpu.async_copy(hr_hbm.at[idxc_v], bufr_v, sem2)
        for g in range(_GCH // 16):
            ir = idxr_v[pl.ds(g * 16, 16)]
            ic = idxc_v[pl.ds(g * 16, 16)]
            rows = g * 16 + lanes
            rx = plsc.load_gather(px_v, [ir]) - plsc.load_gather(px_v, [ic])
            plsc.store_scatter(relbuf_v, [rows, jnp.zeros((16,), jnp.int32)], rx)
            ry = plsc.load_gather(py_v, [ir]) - plsc.load_gather(py_v, [ic])
            plsc.store_scatter(relbuf_v, [rows, jnp.ones((16,), jnp.int32)], ry)
            rz = plsc.load_gather(pz_v, [ir]) - plsc.load_gather(pz_v, [ic])
            plsc.store_scatter(relbuf_v, [rows, jnp.full((16,), 2, jnp.int32)], rz)
        c3 = pltpu.async_copy(relbuf_v, rel_hbm.at[pl.ds(base, _GCH)], sem3)
        c1.wait()
        c2.wait()
        w1 = pltpu.async_copy(bufl_v, hlg_hbm.at[pl.ds(base, _GCH)], sem1)
        w2 = pltpu.async_copy(bufr_v, hrg_hbm.at[pl.ds(base, _GCH)], sem2)
        w1.wait()
        w2.wait()
        c3.wait()
        return carry

    lax.fori_loop(0, n_k, body, 0)


@functools.lru_cache(maxsize=None)
def _make_sc_gather():
    return pl.kernel(
        _sc_gather_body,
        out_type=(
            jax.ShapeDtypeStruct((_E, _H), jnp.float32),
            jax.ShapeDtypeStruct((_E, _H), jnp.float32),
            jax.ShapeDtypeStruct((_E, 16), jnp.float32),
        ),
        mesh=_mesh(),
        compiler_params=pltpu.CompilerParams(needs_layout_passes=False),
        scratch_types=[
            pltpu.VMEM((_GCH,), jnp.int32),
            pltpu.VMEM((_GCH,), jnp.int32),
            pltpu.VMEM((_GCH, _H), jnp.float32),
            pltpu.VMEM((_GCH, _H), jnp.float32),
            pltpu.VMEM((_GCH, 16), jnp.float32),
            pltpu.VMEM((_N,), jnp.float32),
            pltpu.VMEM((_N,), jnp.float32),
            pltpu.VMEM((_N,), jnp.float32),
            pltpu.SemaphoreType.DMA,
            pltpu.SemaphoreType.DMA,
            pltpu.SemaphoreType.DMA,
        ],
    )


def _sc_gather(hl, hr, px, py, pz, row, col):
    return _make_sc_gather()(hl, hr, px, py, pz, row, col)


def _edge_body(hlg_ref, hrg_ref, rel_ref, ea_ref, tm_ref,
               wea_ref, wed_ref, be_ref, wn_ref, bn_ref,
               wi1a_ref, wi1t_ref, bi1_ref, wi2_ref, bi2_ref, out_ref):
    rel = rel_ref[...]
    d2 = jnp.sum(rel * rel, axis=1, keepdims=True)
    dist = jnp.sqrt(d2)
    b = rel.shape[0]
    step = 15.0 / (_DD - 1)
    coeff = -0.5 / (step * step)
    offs = lax.broadcasted_iota(jnp.int32, (b, _DD), 1).astype(jnp.float32) * step
    demb = jnp.exp(coeff * (dist - offs) ** 2)
    ea = (
        jnp.dot(ea_ref[...], wea_ref[...], preferred_element_type=jnp.float32)
        + jnp.dot(demb, wed_ref[...], preferred_element_type=jnp.float32)
        + be_ref[...]
    )
    nf = (
        jnp.dot(hlg_ref[...] * hrg_ref[...], wn_ref[...],
                preferred_element_type=jnp.float32)
        + bn_ref[...]
    )
    x = (
        jnp.dot(ea * nf, wi1a_ref[...], preferred_element_type=jnp.float32)
        + jnp.dot(tm_ref[...], wi1t_ref[...], preferred_element_type=jnp.float32)
        + bi1_ref[...]
    )
    t1 = _silu(x)
    inter = jnp.sum(t1 * wi2_ref[...], axis=1, keepdims=True) + bi2_ref[...]
    w = inter / ((dist + 1.0) * jnp.maximum(dist, 1e-12))
    out_ref[...] = rel * w


def _edge_compute(hlg, hrg, rel, edge_attr, tm,
                  wea, wed, be, wn, bn, wi1a, wi1t, bi1, wi2, bi2):
    be_blk = 2560
    grid = (_E // be_blk,)
    rs = lambda w: pl.BlockSpec((be_blk, w), lambda i: (i, 0))
    full = lambda s: pl.BlockSpec(s, lambda i: (0,) * len(s))
    return pl.pallas_call(
        _edge_body,
        grid=grid,
        in_specs=[
            rs(_H), rs(_H), rs(16), rs(16), rs(16),
            full((16, _H)), full((_DD, _H)), full((1, _H)),
            full((_H, _H)), full((1, _H)),
            full((_H, 2 * _H)), full((16, 2 * _H)), full((1, 2 * _H)),
            full((1, 2 * _H)), full((1, 1)),
        ],
        out_specs=rs(16),
        out_shape=jax.ShapeDtypeStruct((_E, 16), jnp.float32),
    )(hlg, hrg, rel, edge_attr, tm,
      wea, wed, be, wn, bn, wi1a, wi1t, bi1, wi2, bi2)


def _sc_scatter_body(force_hbm, row_hbm, zeros_hbm, out_hbm,
                     val_v, idx_v, accx_v, accy_v, accz_v, sem, sem2):
    c = lax.axis_index("c")
    s = lax.axis_index("s")
    wid = s * _NC + c
    pltpu.sync_copy(zeros_hbm, accx_v)
    pltpu.sync_copy(zeros_hbm, accy_v)
    pltpu.sync_copy(zeros_hbm, accz_v)

    lanes = lax.iota(jnp.int32, 16)
    col0 = jnp.zeros((16,), jnp.int32)
    col1 = jnp.ones((16,), jnp.int32)
    col2 = jnp.full((16,), 2, jnp.int32)

    n_k = _KFULL + jnp.where(wid < _KREM, 1, 0)

    def body(k, carry):
        base = (k * _NW + wid) * _GCH
        i1 = pltpu.async_copy(row_hbm.at[pl.ds(base, _GCH)], idx_v, sem)
        i2 = pltpu.async_copy(force_hbm.at[pl.ds(base, _GCH)], val_v, sem2)
        i1.wait()
        i2.wait()
        for g in range(_GCH // 16):
            rows = g * 16 + lanes
            ir = idx_v[pl.ds(g * 16, 16)]
            fx = plsc.load_gather(val_v, [rows, col0])
            plsc.addupdate_scatter(accx_v, [ir], fx)
            fy = plsc.load_gather(val_v, [rows, col1])
            plsc.addupdate_scatter(accy_v, [ir], fy)
            fz = plsc.load_gather(val_v, [rows, col2])
            plsc.addupdate_scatter(accz_v, [ir], fz)
        return carry

    lax.fori_loop(0, n_k, body, 0)
    obase = wid * 3 * _NPAD
    pltpu.sync_copy(accx_v, out_hbm.at[pl.ds(obase, _NPAD)])
    pltpu.sync_copy(accy_v, out_hbm.at[pl.ds(obase + _NPAD, _NPAD)])
    pltpu.sync_copy(accz_v, out_hbm.at[pl.ds(obase + 2 * _NPAD, _NPAD)])


@functools.lru_cache(maxsize=None)
def _make_sc_scatter():
    return pl.kernel(
        _sc_scatter_body,
        out_type=jax.ShapeDtypeStruct((_NW * 3 * _NPAD,), jnp.float32),
        mesh=_mesh(),
        compiler_params=pltpu.CompilerParams(needs_layout_passes=False),
        scratch_types=[
            pltpu.VMEM((_GCH, 16), jnp.float32),
            pltpu.VMEM((_GCH,), jnp.int32),
            pltpu.VMEM((_NPAD,), jnp.float32),
            pltpu.VMEM((_NPAD,), jnp.float32),
            pltpu.VMEM((_NPAD,), jnp.float32),
            pltpu.SemaphoreType.DMA,
            pltpu.SemaphoreType.DMA,
        ],
    )


def _sc_scatter(force, row, zeros1):
    return _make_sc_scatter()(force, row, zeros1)


def _psum_body(p_ref, out_ref):
    acc = p_ref[0]
    for i in range(1, _NW):
        acc = acc + p_ref[i]
    out_ref[...] = acc


def _psum(p):
    npk = _NPAD // 16
    return pl.pallas_call(
        _psum_body,
        grid=(1,),
        in_specs=[pl.BlockSpec((_NW, npk, 128), lambda i: (0, 0, 0))],
        out_specs=pl.BlockSpec((npk, 128), lambda i: (0, 0)),
        out_shape=jax.ShapeDtypeStruct((npk, 128), jnp.float32),
    )(p)


def _final_body(d_ref, h_ref, ws1h_ref, ws1n_ref, bs1_ref,
                ws2_ref, bs2_ref, out_ref):
    delta = d_ref[...]
    nrm = jnp.sqrt(jnp.sum(delta * delta, axis=1, keepdims=True))
    x = (
        jnp.dot(h_ref[...], ws1h_ref[...], preferred_element_type=jnp.float32)
        + nrm * ws1n_ref[...]
        + bs1_ref[...]
    )
    t = _silu(x)
    g = jnp.sum(t * ws2_ref[...], axis=1, keepdims=True) + bs2_ref[...]
    scale = jax.nn.softplus(g)
    out_ref[...] = delta * scale


def _final(d8, h, ws1h, ws1n, bs1, ws2, bs2):
    bn = 2000
    grid = (_N // bn,)
    rs = lambda w: pl.BlockSpec((bn, w), lambda i: (i, 0))
    full = lambda s: pl.BlockSpec(s, lambda i: (0, 0))
    return pl.pallas_call(
        _final_body,
        grid=grid,
        in_specs=[
            rs(8), rs(_H),
            full((_H, _H // 2)), full((1, _H // 2)), full((1, _H // 2)),
            full((1, _H // 2)), full((1, 1)),
        ],
        out_specs=rs(8),
        out_shape=jax.ShapeDtypeStruct((_N, 8), jnp.float32),
    )(d8, h, ws1h, ws1n, bs1, ws2, bs2)


def kernel(pos, h, edge_index, edge_attr, time,
           W_nl1, b_nl1, W_nl2, b_nl2,
           W_nr1, b_nr1, W_nr2, b_nr2,
           W_e, b_e, W_n, b_n,
           W_i1, b_i1, W_i2, b_i2,
           W_s1, b_s1, W_s2, b_s2):
    row = edge_index[0].astype(jnp.int32)
    col = edge_index[1].astype(jnp.int32)

    px = pos[:, 0]
    py = pos[:, 1]
    pz = pos[:, 2]
    r2 = lambda v: v.reshape(1, -1)

    hl, hr = _node_mlp(h, W_nl1, r2(b_nl1), W_nl2, r2(b_nl2),
                       W_nr1, r2(b_nr1), W_nr2, r2(b_nr2))

    hlg, hrg, rel = _sc_gather(hl, hr, px, py, pz, row, col)

    force = _edge_compute(
        hlg, hrg, rel, edge_attr, time,
        W_e[:16], W_e[16:], r2(b_e), W_n, r2(b_n),
        W_i1[:_H], W_i1[_H:], r2(b_i1), W_i2.reshape(1, -1), b_i2.reshape(1, 1),
    )

    zeros1 = jnp.zeros((_NPAD,), jnp.float32)
    praw = _sc_scatter(force, row, zeros1).reshape(_NW, 3, _NPAD)
    ppk = jnp.pad(praw.transpose(0, 2, 1), ((0, 0), (0, 0), (0, 5)))
    ppk = ppk.reshape(_NW, _NPAD // 16, 128)

    delta8 = _psum(ppk).reshape(_NPAD, 8)[:_N]

    out8 = _final(delta8, h,
                  W_s1[:_H], W_s1[_H:], r2(b_s1),
                  W_s2.reshape(1, -1), b_s2.reshape(1, 1))
    return out8[:, :3]
